# baseline (speedup 1.0000x reference)
import functools

import jax
import jax.numpy as jnp
from jax import lax
from jax.experimental import pallas as pl
from jax.experimental.pallas import tpu as pltpu
from jax.experimental.pallas import tpu_sc as plsc

_NC = 2
_NS = 16
_NW = _NC * _NS

_B = 4096
_S = 200
_D = 32
_BPW = _B // _NW
_HALF = _S // 2

_mesh = plsc.VectorSubcoreMesh(
    core_axis_name="c", subcore_axis_name="s", num_cores=_NC, num_subcores=_NS
)


_NBUF = 8


@functools.partial(
    pl.kernel,
    out_type=jax.ShapeDtypeStruct((_B, _D), jnp.float32),
    mesh=_mesh,
    compiler_params=pltpu.CompilerParams(use_tc_tiling_on_sc=False),
    scratch_types=[
        pltpu.VMEM((_S, _BPW), jnp.int32),
        [pltpu.VMEM((_BPW, _D), jnp.float32) for _ in range(_NBUF)],
        pltpu.VMEM((_BPW, _D), jnp.float32),
        pltpu.SemaphoreType.DMA,
        [pltpu.SemaphoreType.DMA for _ in range(_NBUF)],
    ],
)
def _pool_sum(idx_hbm, table_hbm, out_hbm, idx_v, bufs, acc_v, isem, gsems):
    cid = lax.axis_index("c")
    sid = lax.axis_index("s")
    wid = sid * _NC + cid

    def stage(s, _):
        pltpu.async_copy(idx_hbm.at[s, wid], idx_v.at[s], isem)
        return 0

    lax.fori_loop(0, _S, stage, 0)

    z = jnp.zeros((16,), jnp.float32)

    def zero(j, _):
        acc_v[j, 0:16] = z
        acc_v[j, 16:32] = z
        return 0

    lax.fori_loop(0, _BPW, zero, 0)

    def stage_wait(s, _):
        pltpu.make_async_copy(idx_hbm.at[s, wid], idx_v.at[s], isem).wait()
        return 0

    lax.fori_loop(0, _S, stage_wait, 0)

    def issue(s, b):
        pltpu.async_copy(table_hbm.at[idx_v.at[s]], bufs[b], gsems[b])

    def drain(b):
        pltpu.make_async_copy(table_hbm.at[pl.ds(0, _BPW)], bufs[b], gsems[b]).wait()

    def accum(b):
        buf = bufs[b]

        def red(j, _):
            base = j * 8
            for u in range(8):
                r = base + u
                plsc.addupdate(acc_v.at[r, pl.ds(0, 16)], buf[r, 0:16])
                plsc.addupdate(acc_v.at[r, pl.ds(16, 16)], buf[r, 16:32])
            return 0

        lax.fori_loop(0, _BPW // 8, red, 0)

    for b in range(_NBUF):
        issue(b, b)

    def body(s0, _):
        for b in range(_NBUF):
            s = s0 * _NBUF + b
            drain(b)
            accum(b)
            issue(s + _NBUF, b)
        return 0

    lax.fori_loop(0, (_S - _NBUF) // _NBUF, body, 0)

    for b in range(_NBUF):
        drain(b)
        accum(b)

    pltpu.sync_copy(acc_v, out_hbm.at[pl.ds(wid * _BPW, _BPW)])


_V = 1000000
_TV = 1664
_TG = _TV // 4


def _relayout_body(xt_ref, o_ref):
    x3 = xt_ref[...].reshape(_D, _TG, 4)
    o_ref[...] = jnp.transpose(x3, (1, 2, 0)).reshape(_TG, 128)


def _relayout(tableT):
    return pl.pallas_call(
        _relayout_body,
        grid=((_V + _TV - 1) // _TV,),
        in_specs=[pl.BlockSpec((_D, _TV), lambda i: (0, i))],
        out_specs=pl.BlockSpec((_TG, 128), lambda i: (i, 0)),
        out_shape=jax.ShapeDtypeStruct((_V // 4, 128), jnp.float32),
    )(tableT)


_BB = 512


def _mlp_body(x_ref, w1_ref, b1_ref, w2_ref, b2_ref, o_ref):
    x = x_ref[...] * (1.0 / _S)
    h = jnp.dot(x, w1_ref[...], preferred_element_type=jnp.float32) + b1_ref[...]
    h = jnp.maximum(h, 0.0)
    logits = jnp.dot(h, w2_ref[...], preferred_element_type=jnp.float32) + b2_ref[...]
    m = jnp.max(logits, axis=-1, keepdims=True)
    e = jnp.exp(logits - m)
    o_ref[...] = e / jnp.sum(e, axis=-1, keepdims=True)


def _mlp(pooled_sum, W1, b1, W2, b2):
    n_classes = W2.shape[1]
    hidden = W1.shape[1]
    grid = (_B // _BB,)
    return pl.pallas_call(
        _mlp_body,
        grid=grid,
        in_specs=[
            pl.BlockSpec((_BB, _D), lambda i: (i, 0)),
            pl.BlockSpec((_D, hidden), lambda i: (0, 0)),
            pl.BlockSpec((1, hidden), lambda i: (0, 0)),
            pl.BlockSpec((hidden, n_classes), lambda i: (0, 0)),
            pl.BlockSpec((1, n_classes), lambda i: (0, 0)),
        ],
        out_specs=pl.BlockSpec((_BB, n_classes), lambda i: (i, 0)),
        out_shape=jax.ShapeDtypeStruct((_B, n_classes), jnp.float32),
    )(pooled_sum, W1, b1, W2, b2)


def kernel(inputs, table, W1, b1, W2, b2):
    idx = jnp.swapaxes(inputs.astype(jnp.int32), 0, 1).reshape(_S, _NW, _BPW)
    table_rm = _relayout(jnp.swapaxes(table, 0, 1)).reshape(_V, _D)
    pooled_sum = _pool_sum(idx, table_rm)
    return _mlp(pooled_sum, W1, b1.reshape(1, -1), W2, b2.reshape(1, -1))


# speedup vs baseline: 5.0129x; 5.0129x over previous
import functools

import jax
import jax.numpy as jnp
from jax import lax
from jax.experimental import pallas as pl
from jax.experimental.pallas import tpu as pltpu
from jax.experimental.pallas import tpu_sc as plsc

_NC = 2
_NS = 16
_NW = _NC * _NS

_B = 4096
_S = 200
_D = 32
_BPW = _B // _NW
_HALF = _S // 2

_mesh = plsc.VectorSubcoreMesh(
    core_axis_name="c", subcore_axis_name="s", num_cores=_NC, num_subcores=_NS
)


_NBUF = 8


@functools.partial(
    pl.kernel,
    out_type=jax.ShapeDtypeStruct((_B, _D), jnp.float32),
    mesh=_mesh,
    compiler_params=pltpu.CompilerParams(use_tc_tiling_on_sc=False),
    scratch_types=[
        pltpu.VMEM((_S, _BPW), jnp.int32),
        [pltpu.VMEM((_BPW, _D), jnp.float32) for _ in range(_NBUF)],
        pltpu.VMEM((_BPW, _D), jnp.float32),
        pltpu.SemaphoreType.DMA,
        [pltpu.SemaphoreType.DMA for _ in range(_NBUF)],
    ],
)
def _pool_sum(idx_hbm, table_hbm, out_hbm, idx_v, bufs, acc_v, isem, gsems):
    cid = lax.axis_index("c")
    sid = lax.axis_index("s")
    wid = sid * _NC + cid

    def stage(s, _):
        pltpu.async_copy(idx_hbm.at[s, wid], idx_v.at[s], isem)
        return 0

    lax.fori_loop(0, _S, stage, 0)

    z = jnp.zeros((16,), jnp.float32)

    def zero(j, _):
        acc_v[j, 0:16] = z
        acc_v[j, 16:32] = z
        return 0

    lax.fori_loop(0, _BPW, zero, 0)

    def stage_wait(s, _):
        pltpu.make_async_copy(idx_hbm.at[s, wid], idx_v.at[s], isem).wait()
        return 0

    lax.fori_loop(0, _S, stage_wait, 0)

    def issue(s, b):
        pltpu.async_copy(table_hbm.at[idx_v.at[s]], bufs[b], gsems[b])

    def drain(b):
        pltpu.make_async_copy(table_hbm.at[pl.ds(0, _BPW)], bufs[b], gsems[b]).wait()

    def accum(b):
        buf = bufs[b]

        def red(j, _):
            base = j * 8
            for u in range(8):
                r = base + u
                plsc.addupdate(acc_v.at[r, pl.ds(0, 16)], buf[r, 0:16])
                plsc.addupdate(acc_v.at[r, pl.ds(16, 16)], buf[r, 16:32])
            return 0

        lax.fori_loop(0, _BPW // 8, red, 0)

    for b in range(_NBUF):
        issue(b, b)

    def body(s0, _):
        for b in range(_NBUF):
            s = s0 * _NBUF + b
            drain(b)
            accum(b)
            issue(s + _NBUF, b)
        return 0

    lax.fori_loop(0, (_S - _NBUF) // _NBUF, body, 0)

    for b in range(_NBUF):
        drain(b)
        accum(b)

    pltpu.sync_copy(acc_v, out_hbm.at[pl.ds(wid * _BPW, _BPW)])


_V = 1000000
_TV = 2048
_TG = _TV // 4
_NMAIN = _V // _TV
_VTAIL = _V - _NMAIN * _TV


def _relayout_body(xt_ref, o_ref):
    y = xt_ref[...].T
    for q in range(4):
        o_ref[:, 32 * q : 32 * (q + 1)] = y[_TG * q : _TG * (q + 1), :]


def _tail_body(alias_ref, xt_ref, o_ref):
    del alias_ref
    x3 = xt_ref[0].reshape(_D, 16, 4)
    o_ref[...] = jnp.transpose(x3, (1, 2, 0)).reshape(16, 128)


def _relayout(tableT):
    main = pl.pallas_call(
        _relayout_body,
        grid=(_NMAIN,),
        in_specs=[pl.BlockSpec((_D, _TV), lambda i: (0, i))],
        out_specs=pl.BlockSpec((_TG, 128), lambda i: (i, 0)),
        out_shape=jax.ShapeDtypeStruct((_V // 4, 128), jnp.float32),
    )(tableT)
    xt_tail = lax.slice(tableT, (0, _NMAIN * _TV), (_D, _V))
    xt_tail_r = jnp.swapaxes(xt_tail.reshape(_D, _VTAIL // 64, 64), 0, 1)
    base_blk = _NMAIN * _TG // 16
    return pl.pallas_call(
        _tail_body,
        grid=(_VTAIL // 64,),
        in_specs=[
            pl.BlockSpec(memory_space=pl.ANY),
            pl.BlockSpec((1, _D, 64), lambda i: (i, 0, 0)),
        ],
        out_specs=pl.BlockSpec((16, 128), lambda i: (base_blk + i, 0)),
        out_shape=jax.ShapeDtypeStruct((_V // 4, 128), jnp.float32),
        input_output_aliases={0: 0},
    )(main, xt_tail_r)


def _permute_idx(i):
    m = i & (_TV - 1)
    r_main = (i - m) + 4 * (m & (_TG - 1)) + (m >> 9)
    return jnp.where(i >= _NMAIN * _TV, i, r_main)


_BB = 512


def _mlp_body(x_ref, w1_ref, b1_ref, w2_ref, b2_ref, o_ref):
    x = x_ref[...] * (1.0 / _S)
    h = jnp.dot(x, w1_ref[...], preferred_element_type=jnp.float32) + b1_ref[...]
    h = jnp.maximum(h, 0.0)
    logits = jnp.dot(h, w2_ref[...], preferred_element_type=jnp.float32) + b2_ref[...]
    m = jnp.max(logits, axis=-1, keepdims=True)
    e = jnp.exp(logits - m)
    o_ref[...] = e / jnp.sum(e, axis=-1, keepdims=True)


def _mlp(pooled_sum, W1, b1, W2, b2):
    n_classes = W2.shape[1]
    hidden = W1.shape[1]
    grid = (_B // _BB,)
    return pl.pallas_call(
        _mlp_body,
        grid=grid,
        in_specs=[
            pl.BlockSpec((_BB, _D), lambda i: (i, 0)),
            pl.BlockSpec((_D, hidden), lambda i: (0, 0)),
            pl.BlockSpec((1, hidden), lambda i: (0, 0)),
            pl.BlockSpec((hidden, n_classes), lambda i: (0, 0)),
            pl.BlockSpec((1, n_classes), lambda i: (0, 0)),
        ],
        out_specs=pl.BlockSpec((_BB, n_classes), lambda i: (i, 0)),
        out_shape=jax.ShapeDtypeStruct((_B, n_classes), jnp.float32),
    )(pooled_sum, W1, b1, W2, b2)


def kernel(inputs, table, W1, b1, W2, b2):
    idx = _permute_idx(
        jnp.swapaxes(inputs.astype(jnp.int32), 0, 1).reshape(_S, _NW, _BPW)
    )
    table_rm = _relayout(jnp.swapaxes(table, 0, 1)).reshape(_V, _D)
    pooled_sum = _pool_sum(idx, table_rm)
    return _mlp(pooled_sum, W1, b1.reshape(1, -1), W2, b2.reshape(1, -1))


# speedup vs baseline: 7.3233x; 1.4609x over previous
import functools

import jax
import jax.numpy as jnp
from jax import lax
from jax.experimental import pallas as pl
from jax.experimental.pallas import tpu as pltpu
from jax.experimental.pallas import tpu_sc as plsc

_NC = 2
_NS = 16
_NW = _NC * _NS

_B = 4096
_S = 200
_D = 32
_BPW = _B // _NW
_HALF = _S // 2

_mesh = plsc.VectorSubcoreMesh(
    core_axis_name="c", subcore_axis_name="s", num_cores=_NC, num_subcores=_NS
)


_NBUF = 8


@functools.partial(
    pl.kernel,
    out_type=jax.ShapeDtypeStruct((_B, _D), jnp.float32),
    mesh=_mesh,
    compiler_params=pltpu.CompilerParams(use_tc_tiling_on_sc=False),
    scratch_types=[
        pltpu.VMEM((_S, _BPW), jnp.int32),
        [pltpu.VMEM((_BPW, _D), jnp.float32) for _ in range(_NBUF)],
        pltpu.VMEM((_BPW, _D), jnp.float32),
        pltpu.SemaphoreType.DMA,
        [pltpu.SemaphoreType.DMA for _ in range(_NBUF)],
    ],
)
def _pool_sum(idx_hbm, table_hbm, out_hbm, idx_v, bufs, acc_v, isem, gsems):
    cid = lax.axis_index("c")
    sid = lax.axis_index("s")
    wid = sid * _NC + cid

    def stage(s, _):
        pltpu.async_copy(idx_hbm.at[s, wid], idx_v.at[s], isem)
        return 0

    lax.fori_loop(0, _S, stage, 0)

    z = jnp.zeros((16,), jnp.float32)

    def zero(j, _):
        acc_v[j, 0:16] = z
        acc_v[j, 16:32] = z
        return 0

    lax.fori_loop(0, _BPW, zero, 0)

    def stage_wait(s, _):
        pltpu.make_async_copy(idx_hbm.at[s, wid], idx_v.at[s], isem).wait()
        return 0

    lax.fori_loop(0, _S, stage_wait, 0)

    def issue(s, b):
        pltpu.async_copy(table_hbm.at[idx_v.at[s]], bufs[b], gsems[b])

    def drain(b):
        pltpu.make_async_copy(table_hbm.at[pl.ds(0, _BPW)], bufs[b], gsems[b]).wait()

    def accum(b):
        buf = bufs[b]

        def red(j, _):
            base = j * 8
            for u in range(8):
                r = base + u
                plsc.addupdate(acc_v.at[r, pl.ds(0, 16)], buf[r, 0:16])
                plsc.addupdate(acc_v.at[r, pl.ds(16, 16)], buf[r, 16:32])
            return 0

        lax.fori_loop(0, _BPW // 8, red, 0)

    for b in range(_NBUF):
        issue(b, b)

    def body(s0, _):
        for b in range(_NBUF):
            s = s0 * _NBUF + b
            drain(b)
            accum(b)
            issue(s + _NBUF, b)
        return 0

    lax.fori_loop(0, (_S - _NBUF) // _NBUF, body, 0)

    for b in range(_NBUF):
        drain(b)
        accum(b)

    pltpu.sync_copy(acc_v, out_hbm.at[pl.ds(wid * _BPW, _BPW)])


_V = 1000000
_TV = 8192
_TG = _TV // 4
_NMAIN = _V // _TV
_VTAIL = _V - _NMAIN * _TV


def _relayout_body(xt_ref, o_ref):
    eye = jnp.eye(_D, dtype=jnp.float32)
    y = jax.lax.dot_general(
        xt_ref[...], eye, (((0,), (0,)), ((), ())),
        preferred_element_type=jnp.float32,
    )
    for q in range(4):
        o_ref[:, 32 * q : 32 * (q + 1)] = y[_TG * q : _TG * (q + 1), :]


def _tail_body(alias_ref, xt_ref, o_ref):
    del alias_ref
    x3 = xt_ref[0].reshape(_D, 16, 4)
    o_ref[...] = jnp.transpose(x3, (1, 2, 0)).reshape(16, 128)


def _relayout(tableT):
    main = pl.pallas_call(
        _relayout_body,
        grid=(_NMAIN,),
        in_specs=[pl.BlockSpec((_D, _TV), lambda i: (0, i))],
        out_specs=pl.BlockSpec((_TG, 128), lambda i: (i, 0)),
        out_shape=jax.ShapeDtypeStruct((_V // 4, 128), jnp.float32),
    )(tableT)
    xt_tail = lax.slice(tableT, (0, _NMAIN * _TV), (_D, _V))
    xt_tail_r = jnp.swapaxes(xt_tail.reshape(_D, _VTAIL // 64, 64), 0, 1)
    base_blk = _NMAIN * _TG // 16
    return pl.pallas_call(
        _tail_body,
        grid=(_VTAIL // 64,),
        in_specs=[
            pl.BlockSpec(memory_space=pl.ANY),
            pl.BlockSpec((1, _D, 64), lambda i: (i, 0, 0)),
        ],
        out_specs=pl.BlockSpec((16, 128), lambda i: (base_blk + i, 0)),
        out_shape=jax.ShapeDtypeStruct((_V // 4, 128), jnp.float32),
        input_output_aliases={0: 0},
    )(main, xt_tail_r)


def _permute_idx(i):
    m = i & (_TV - 1)
    r_main = (i - m) + 4 * (m & (_TG - 1)) + (m >> 11)
    return jnp.where(i >= _NMAIN * _TV, i, r_main)


_BB = 512


def _mlp_body(x_ref, w1_ref, b1_ref, w2_ref, b2_ref, o_ref):
    x = x_ref[...] * (1.0 / _S)
    h = jnp.dot(x, w1_ref[...], preferred_element_type=jnp.float32) + b1_ref[...]
    h = jnp.maximum(h, 0.0)
    logits = jnp.dot(h, w2_ref[...], preferred_element_type=jnp.float32) + b2_ref[...]
    m = jnp.max(logits, axis=-1, keepdims=True)
    e = jnp.exp(logits - m)
    o_ref[...] = e / jnp.sum(e, axis=-1, keepdims=True)


def _mlp(pooled_sum, W1, b1, W2, b2):
    n_classes = W2.shape[1]
    hidden = W1.shape[1]
    grid = (_B // _BB,)
    return pl.pallas_call(
        _mlp_body,
        grid=grid,
        in_specs=[
            pl.BlockSpec((_BB, _D), lambda i: (i, 0)),
            pl.BlockSpec((_D, hidden), lambda i: (0, 0)),
            pl.BlockSpec((1, hidden), lambda i: (0, 0)),
            pl.BlockSpec((hidden, n_classes), lambda i: (0, 0)),
            pl.BlockSpec((1, n_classes), lambda i: (0, 0)),
        ],
        out_specs=pl.BlockSpec((_BB, n_classes), lambda i: (i, 0)),
        out_shape=jax.ShapeDtypeStruct((_B, n_classes), jnp.float32),
    )(pooled_sum, W1, b1, W2, b2)


def kernel(inputs, table, W1, b1, W2, b2):
    idx = _permute_idx(
        jnp.swapaxes(inputs.astype(jnp.int32), 0, 1).reshape(_S, _NW, _BPW)
    )
    table_rm = _relayout(jnp.swapaxes(table, 0, 1)).reshape(_V, _D)
    pooled_sum = _pool_sum(idx, table_rm)
    return _mlp(pooled_sum, W1, b1.reshape(1, -1), W2, b2.reshape(1, -1))


# speedup vs baseline: 7.4683x; 1.0198x over previous
import functools

import jax
import jax.numpy as jnp
from jax import lax
from jax.experimental import pallas as pl
from jax.experimental.pallas import tpu as pltpu
from jax.experimental.pallas import tpu_sc as plsc

_NC = 2
_NS = 16
_NW = _NC * _NS

_B = 4096
_S = 200
_D = 32
_BPW = _B // _NW
_HALF = _S // 2

_mesh = plsc.VectorSubcoreMesh(
    core_axis_name="c", subcore_axis_name="s", num_cores=_NC, num_subcores=_NS
)


_NBUF = 8


@functools.partial(
    pl.kernel,
    out_type=jax.ShapeDtypeStruct((_B, _D), jnp.float32),
    mesh=_mesh,
    compiler_params=pltpu.CompilerParams(use_tc_tiling_on_sc=False),
    scratch_types=[
        pltpu.VMEM((_S, _BPW), jnp.int32),
        [pltpu.VMEM((_BPW, _D), jnp.float32) for _ in range(_NBUF)],
        pltpu.VMEM((_BPW, _D), jnp.float32),
        pltpu.SemaphoreType.DMA,
        [pltpu.SemaphoreType.DMA for _ in range(_NBUF)],
    ],
)
def _pool_sum(idx_hbm, table_hbm, out_hbm, idx_v, bufs, acc_v, isem, gsems):
    cid = lax.axis_index("c")
    sid = lax.axis_index("s")
    wid = sid * _NC + cid

    stage_cp = pltpu.async_copy(idx_hbm.at[:, wid], idx_v, isem)

    z = jnp.zeros((16,), jnp.float32)

    def zero(j, _):
        acc_v[j, 0:16] = z
        acc_v[j, 16:32] = z
        return 0

    lax.fori_loop(0, _BPW, zero, 0)
    stage_cp.wait()

    def issue(s, b):
        pltpu.async_copy(table_hbm.at[idx_v.at[s]], bufs[b], gsems[b])

    def drain(b):
        pltpu.make_async_copy(table_hbm.at[pl.ds(0, _BPW)], bufs[b], gsems[b]).wait()

    def accum(b):
        buf = bufs[b]

        def red(j, _):
            base = j * 8
            for u in range(8):
                r = base + u
                plsc.addupdate(acc_v.at[r, pl.ds(0, 16)], buf[r, 0:16])
                plsc.addupdate(acc_v.at[r, pl.ds(16, 16)], buf[r, 16:32])
            return 0

        lax.fori_loop(0, _BPW // 8, red, 0)

    for b in range(_NBUF):
        issue(b, b)

    def body(s0, _):
        for b in range(_NBUF):
            s = s0 * _NBUF + b
            drain(b)
            accum(b)
            issue(s + _NBUF, b)
        return 0

    lax.fori_loop(0, (_S - _NBUF) // _NBUF, body, 0)

    for b in range(_NBUF):
        drain(b)
        accum(b)

    pltpu.sync_copy(acc_v, out_hbm.at[pl.ds(wid * _BPW, _BPW)])


_V = 1000000
_TV = 16384
_TG = _TV // 4
_NMAIN = _V // _TV
_VTAIL = _V - _NMAIN * _TV


def _relayout_body(xt_ref, o_ref):
    eye = jnp.eye(_D, dtype=jnp.float32)
    for q in range(4):
        yq = jax.lax.dot_general(
            xt_ref[:, _TG * q : _TG * (q + 1)], eye, (((0,), (0,)), ((), ())),
            preferred_element_type=jnp.float32,
        )
        o_ref[:, 32 * q : 32 * (q + 1)] = yq


def _tail_body(alias_ref, xt_ref, o_ref):
    del alias_ref
    x3 = xt_ref[0].reshape(_D, 16, 4)
    o_ref[...] = jnp.transpose(x3, (1, 2, 0)).reshape(16, 128)


def _relayout(tableT):
    main = pl.pallas_call(
        _relayout_body,
        grid=(_NMAIN,),
        in_specs=[pl.BlockSpec((_D, _TV), lambda i: (0, i))],
        out_specs=pl.BlockSpec((_TG, 128), lambda i: (i, 0)),
        out_shape=jax.ShapeDtypeStruct((_V // 4, 128), jnp.float32),
    )(tableT)
    xt_tail = lax.slice(tableT, (0, _NMAIN * _TV), (_D, _V))
    xt_tail_r = jnp.swapaxes(xt_tail.reshape(_D, _VTAIL // 64, 64), 0, 1)
    base_blk = _NMAIN * _TG // 16
    return pl.pallas_call(
        _tail_body,
        grid=(_VTAIL // 64,),
        in_specs=[
            pl.BlockSpec(memory_space=pl.ANY),
            pl.BlockSpec((1, _D, 64), lambda i: (i, 0, 0)),
        ],
        out_specs=pl.BlockSpec((16, 128), lambda i: (base_blk + i, 0)),
        out_shape=jax.ShapeDtypeStruct((_V // 4, 128), jnp.float32),
        input_output_aliases={0: 0},
    )(main, xt_tail_r)


def _permute_idx(i):
    m = i & (_TV - 1)
    r_main = (i - m) + 4 * (m & (_TG - 1)) + (m >> 12)
    return jnp.where(i >= _NMAIN * _TV, i, r_main)


_BB = 512


def _mlp_body(x_ref, w1_ref, b1_ref, w2_ref, b2_ref, o_ref):
    x = x_ref[...] * (1.0 / _S)
    h = jnp.dot(x, w1_ref[...], preferred_element_type=jnp.float32) + b1_ref[...]
    h = jnp.maximum(h, 0.0)
    logits = jnp.dot(h, w2_ref[...], preferred_element_type=jnp.float32) + b2_ref[...]
    m = jnp.max(logits, axis=-1, keepdims=True)
    e = jnp.exp(logits - m)
    o_ref[...] = e / jnp.sum(e, axis=-1, keepdims=True)


def _mlp(pooled_sum, W1, b1, W2, b2):
    n_classes = W2.shape[1]
    hidden = W1.shape[1]
    grid = (_B // _BB,)
    return pl.pallas_call(
        _mlp_body,
        grid=grid,
        in_specs=[
            pl.BlockSpec((_BB, _D), lambda i: (i, 0)),
            pl.BlockSpec((_D, hidden), lambda i: (0, 0)),
            pl.BlockSpec((1, hidden), lambda i: (0, 0)),
            pl.BlockSpec((hidden, n_classes), lambda i: (0, 0)),
            pl.BlockSpec((1, n_classes), lambda i: (0, 0)),
        ],
        out_specs=pl.BlockSpec((_BB, n_classes), lambda i: (i, 0)),
        out_shape=jax.ShapeDtypeStruct((_B, n_classes), jnp.float32),
    )(pooled_sum, W1, b1, W2, b2)


def kernel(inputs, table, W1, b1, W2, b2):
    idx = _permute_idx(
        jnp.swapaxes(inputs.astype(jnp.int32), 0, 1).reshape(_S, _NW, _BPW)
    )
    table_rm = _relayout(jnp.swapaxes(table, 0, 1)).reshape(_V, _D)
    pooled_sum = _pool_sum(idx, table_rm)
    return _mlp(pooled_sum, W1, b1.reshape(1, -1), W2, b2.reshape(1, -1))


# speedup vs baseline: 10.0705x; 1.3484x over previous
import functools

import jax
import jax.numpy as jnp
from jax import lax
from jax.experimental import pallas as pl
from jax.experimental.pallas import tpu as pltpu
from jax.experimental.pallas import tpu_sc as plsc

_NC = 2
_NS = 16
_NW = _NC * _NS

_B = 4096
_S = 200
_D = 32
_BPW = _B // _NW
_HALF = _S // 2

_mesh = plsc.VectorSubcoreMesh(
    core_axis_name="c", subcore_axis_name="s", num_cores=_NC, num_subcores=_NS
)


_NBUF = 8


@functools.partial(
    pl.kernel,
    out_type=jax.ShapeDtypeStruct((_B, _D), jnp.float32),
    mesh=_mesh,
    compiler_params=pltpu.CompilerParams(use_tc_tiling_on_sc=False),
    scratch_types=[
        pltpu.VMEM((_S, _BPW), jnp.int32),
        [pltpu.VMEM((_BPW, _D), jnp.float32) for _ in range(_NBUF)],
        pltpu.VMEM((_BPW, _D), jnp.float32),
        pltpu.SemaphoreType.DMA,
        [pltpu.SemaphoreType.DMA for _ in range(_NBUF)],
    ],
)
def _pool_sum(idx_hbm, table_hbm, out_hbm, idx_v, bufs, acc_v, isem, gsems):
    cid = lax.axis_index("c")
    sid = lax.axis_index("s")
    wid = sid * _NC + cid

    stage_cp = pltpu.async_copy(idx_hbm.at[:, wid], idx_v, isem)

    z = jnp.zeros((16,), jnp.float32)

    def zero(j, _):
        acc_v[j, 0:16] = z
        acc_v[j, 16:32] = z
        return 0

    lax.fori_loop(0, _BPW, zero, 0)
    stage_cp.wait()

    def issue(s, b):
        pltpu.async_copy(table_hbm.at[idx_v.at[s]], bufs[b], gsems[b])

    def drain(b):
        pltpu.make_async_copy(table_hbm.at[pl.ds(0, _BPW)], bufs[b], gsems[b]).wait()

    def accum(b):
        buf = bufs[b]

        def red(j, _):
            base = j * 8
            for u in range(8):
                r = base + u
                plsc.addupdate(acc_v.at[r, pl.ds(0, 16)], buf[r, 0:16])
                plsc.addupdate(acc_v.at[r, pl.ds(16, 16)], buf[r, 16:32])
            return 0

        lax.fori_loop(0, _BPW // 8, red, 0)

    for b in range(_NBUF):
        issue(b, b)

    def body(s0, _):
        for b in range(_NBUF):
            s = s0 * _NBUF + b
            drain(b)
            accum(b)
            issue(s + _NBUF, b)
        return 0

    lax.fori_loop(0, (_S - _NBUF) // _NBUF, body, 0)

    for b in range(_NBUF):
        drain(b)
        accum(b)

    pltpu.sync_copy(acc_v, out_hbm.at[pl.ds(wid * _BPW, _BPW)])


_V = 1000000
_TV = 16384
_TG = _TV // 4
_NMAIN = _V // _TV
_VTAIL = _V - _NMAIN * _TV


def _relayout_body(xt_ref, o_ref):
    eye128 = jnp.eye(128, dtype=jnp.float32)
    acc = None
    for q in range(4):
        eq = lax.slice(eye128, (32 * q, 0), (32 * (q + 1), 128))
        yq = jax.lax.dot_general(
            xt_ref[:, _TG * q : _TG * (q + 1)], eq, (((0,), (0,)), ((), ())),
            preferred_element_type=jnp.float32,
        )
        acc = yq if acc is None else acc + yq
    o_ref[...] = acc


def _tail_body(alias_ref, xt_ref, o_ref):
    del alias_ref
    x3 = xt_ref[0].reshape(_D, 16, 4)
    o_ref[...] = jnp.transpose(x3, (1, 2, 0)).reshape(16, 128)


def _relayout(tableT):
    main = pl.pallas_call(
        _relayout_body,
        grid=(_NMAIN,),
        in_specs=[pl.BlockSpec((_D, _TV), lambda i: (0, i))],
        out_specs=pl.BlockSpec((_TG, 128), lambda i: (i, 0)),
        out_shape=jax.ShapeDtypeStruct((_V // 4, 128), jnp.float32),
    )(tableT)
    xt_tail = lax.slice(tableT, (0, _NMAIN * _TV), (_D, _V))
    xt_tail_r = jnp.swapaxes(xt_tail.reshape(_D, _VTAIL // 64, 64), 0, 1)
    base_blk = _NMAIN * _TG // 16
    return pl.pallas_call(
        _tail_body,
        grid=(_VTAIL // 64,),
        in_specs=[
            pl.BlockSpec(memory_space=pl.ANY),
            pl.BlockSpec((1, _D, 64), lambda i: (i, 0, 0)),
        ],
        out_specs=pl.BlockSpec((16, 128), lambda i: (base_blk + i, 0)),
        out_shape=jax.ShapeDtypeStruct((_V // 4, 128), jnp.float32),
        input_output_aliases={0: 0},
    )(main, xt_tail_r)


def _permute_idx(i):
    m = i & (_TV - 1)
    r_main = (i - m) + 4 * (m & (_TG - 1)) + (m >> 12)
    return jnp.where(i >= _NMAIN * _TV, i, r_main)


_BB = 512


def _mlp_body(x_ref, w1_ref, b1_ref, w2_ref, b2_ref, o_ref):
    x = x_ref[...] * (1.0 / _S)
    h = jnp.dot(x, w1_ref[...], preferred_element_type=jnp.float32) + b1_ref[...]
    h = jnp.maximum(h, 0.0)
    logits = jnp.dot(h, w2_ref[...], preferred_element_type=jnp.float32) + b2_ref[...]
    m = jnp.max(logits, axis=-1, keepdims=True)
    e = jnp.exp(logits - m)
    o_ref[...] = e / jnp.sum(e, axis=-1, keepdims=True)


def _mlp(pooled_sum, W1, b1, W2, b2):
    n_classes = W2.shape[1]
    hidden = W1.shape[1]
    grid = (_B // _BB,)
    return pl.pallas_call(
        _mlp_body,
        grid=grid,
        in_specs=[
            pl.BlockSpec((_BB, _D), lambda i: (i, 0)),
            pl.BlockSpec((_D, hidden), lambda i: (0, 0)),
            pl.BlockSpec((1, hidden), lambda i: (0, 0)),
            pl.BlockSpec((hidden, n_classes), lambda i: (0, 0)),
            pl.BlockSpec((1, n_classes), lambda i: (0, 0)),
        ],
        out_specs=pl.BlockSpec((_BB, n_classes), lambda i: (i, 0)),
        out_shape=jax.ShapeDtypeStruct((_B, n_classes), jnp.float32),
    )(pooled_sum, W1, b1, W2, b2)


def kernel(inputs, table, W1, b1, W2, b2):
    idx = _permute_idx(
        jnp.swapaxes(inputs.astype(jnp.int32), 0, 1).reshape(_S, _NW, _BPW)
    )
    table_rm = _relayout(jnp.swapaxes(table, 0, 1)).reshape(_V, _D)
    pooled_sum = _pool_sum(idx, table_rm)
    return _mlp(pooled_sum, W1, b1.reshape(1, -1), W2, b2.reshape(1, -1))


# speedup vs baseline: 10.1064x; 1.0036x over previous
import functools

import jax
import jax.numpy as jnp
from jax import lax
from jax.experimental import pallas as pl
from jax.experimental.pallas import tpu as pltpu
from jax.experimental.pallas import tpu_sc as plsc

_NC = 2
_NS = 16
_NW = _NC * _NS

_B = 4096
_S = 200
_D = 32
_BPW = _B // _NW

_mesh = plsc.VectorSubcoreMesh(
    core_axis_name="c", subcore_axis_name="s", num_cores=_NC, num_subcores=_NS
)


_NBUF = 8


@functools.partial(
    pl.kernel,
    out_type=jax.ShapeDtypeStruct((_B, _D), jnp.float32),
    mesh=_mesh,
    compiler_params=pltpu.CompilerParams(use_tc_tiling_on_sc=False),
    scratch_types=[
        pltpu.VMEM((_S, _BPW), jnp.int32),
        [pltpu.VMEM((_BPW, _D), jnp.float32) for _ in range(_NBUF)],
        pltpu.VMEM((_BPW, _D), jnp.float32),
        pltpu.SemaphoreType.DMA,
        [pltpu.SemaphoreType.DMA for _ in range(_NBUF)],
    ],
)
def _pool_sum(idx_hbm, table_hbm, out_hbm, idx_v, bufs, acc_v, isem, gsems):
    cid = lax.axis_index("c")
    sid = lax.axis_index("s")
    wid = sid * _NC + cid

    stage_cp = pltpu.async_copy(idx_hbm.at[:, wid], idx_v, isem)

    z = jnp.zeros((16,), jnp.float32)

    def zero(j, _):
        acc_v[j, 0:16] = z
        acc_v[j, 16:32] = z
        return 0

    lax.fori_loop(0, _BPW, zero, 0)
    stage_cp.wait()

    def issue(s, b):
        pltpu.async_copy(table_hbm.at[idx_v.at[s]], bufs[b], gsems[b])

    def drain(b):
        pltpu.make_async_copy(table_hbm.at[pl.ds(0, _BPW)], bufs[b], gsems[b]).wait()

    def accum(b):
        buf = bufs[b]

        def red(j, _):
            base = j * 8
            for u in range(8):
                r = base + u
                plsc.addupdate(acc_v.at[r, pl.ds(0, 16)], buf[r, 0:16])
                plsc.addupdate(acc_v.at[r, pl.ds(16, 16)], buf[r, 16:32])
            return 0

        lax.fori_loop(0, _BPW // 8, red, 0)

    for b in range(_NBUF):
        issue(b, b)

    def body(s0, _):
        for b in range(_NBUF):
            s = s0 * _NBUF + b
            drain(b)
            accum(b)
            issue(s + _NBUF, b)
        return 0

    lax.fori_loop(0, (_S - _NBUF) // _NBUF, body, 0)

    for b in range(_NBUF):
        drain(b)
        accum(b)

    pltpu.sync_copy(acc_v, out_hbm.at[pl.ds(wid * _BPW, _BPW)])


_V = 1000000
_TV = 16384
_TG = _TV // 4
_NMAIN = _V // _TV
_VTAIL = _V - _NMAIN * _TV


def _relayout_body(xt_ref, o_ref):
    eye128 = jnp.eye(128, dtype=jnp.float32)
    acc = None
    for q in range(4):
        eq = lax.slice(eye128, (32 * q, 0), (32 * (q + 1), 128))
        yq = jax.lax.dot_general(
            xt_ref[:, _TG * q : _TG * (q + 1)], eq, (((0,), (0,)), ((), ())),
            preferred_element_type=jnp.float32,
        )
        acc = yq if acc is None else acc + yq
    o_ref[...] = acc


def _tail_body(alias_ref, xt_ref, o_ref):
    del alias_ref
    x3 = xt_ref[0].reshape(_D, 16, 4)
    o_ref[...] = jnp.transpose(x3, (1, 2, 0)).reshape(16, 128)


def _relayout(tableT):
    main = pl.pallas_call(
        _relayout_body,
        grid=(_NMAIN,),
        in_specs=[pl.BlockSpec((_D, _TV), lambda i: (0, i))],
        out_specs=pl.BlockSpec((_TG, 128), lambda i: (i, 0)),
        out_shape=jax.ShapeDtypeStruct((_V // 4, 128), jnp.float32),
    )(tableT)
    xt_tail = lax.slice(tableT, (0, _NMAIN * _TV), (_D, _V))
    xt_tail_r = jnp.swapaxes(xt_tail.reshape(_D, _VTAIL // 64, 64), 0, 1)
    base_blk = _NMAIN * _TG // 16
    return pl.pallas_call(
        _tail_body,
        grid=(_VTAIL // 64,),
        in_specs=[
            pl.BlockSpec(memory_space=pl.ANY),
            pl.BlockSpec((1, _D, 64), lambda i: (i, 0, 0)),
        ],
        out_specs=pl.BlockSpec((16, 128), lambda i: (base_blk + i, 0)),
        out_shape=jax.ShapeDtypeStruct((_V // 4, 128), jnp.float32),
        input_output_aliases={0: 0},
    )(main, xt_tail_r)


def _permute_idx(i):
    m = i & (_TV - 1)
    r_main = (i - m) + 4 * (m & (_TG - 1)) + (m >> 12)
    return jnp.where(i >= _NMAIN * _TV, i, r_main)


_BB = 512


def _mlp_body(x_ref, w1_ref, b1_ref, w2_ref, b2_ref, o_ref):
    x = x_ref[...] * (1.0 / _S)
    h = jnp.dot(x, w1_ref[...], preferred_element_type=jnp.float32) + b1_ref[...]
    h = jnp.maximum(h, 0.0)
    logits = jnp.dot(h, w2_ref[...], preferred_element_type=jnp.float32) + b2_ref[...]
    m = jnp.max(logits, axis=-1, keepdims=True)
    e = jnp.exp(logits - m)
    o_ref[...] = e / jnp.sum(e, axis=-1, keepdims=True)


def _mlp(pooled_sum, W1, b1, W2, b2):
    n_classes = W2.shape[1]
    hidden = W1.shape[1]
    grid = (_B // _BB,)
    return pl.pallas_call(
        _mlp_body,
        grid=grid,
        in_specs=[
            pl.BlockSpec((_BB, _D), lambda i: (i, 0)),
            pl.BlockSpec((_D, hidden), lambda i: (0, 0)),
            pl.BlockSpec((1, hidden), lambda i: (0, 0)),
            pl.BlockSpec((hidden, n_classes), lambda i: (0, 0)),
            pl.BlockSpec((1, n_classes), lambda i: (0, 0)),
        ],
        out_specs=pl.BlockSpec((_BB, n_classes), lambda i: (i, 0)),
        out_shape=jax.ShapeDtypeStruct((_B, n_classes), jnp.float32),
    )(pooled_sum, W1, b1, W2, b2)


def kernel(inputs, table, W1, b1, W2, b2):
    idx = _permute_idx(
        jnp.swapaxes(inputs.astype(jnp.int32), 0, 1).reshape(_S, _NW, _BPW)
    )
    table_rm = _relayout(jnp.swapaxes(table, 0, 1)).reshape(_V, _D)
    pooled_sum = _pool_sum(idx, table_rm)
    return _mlp(pooled_sum, W1, b1.reshape(1, -1), W2, b2.reshape(1, -1))
